# baseline (speedup 1.0000x reference)
import functools

import jax
import jax.numpy as jnp
from jax import lax
from jax.experimental import pallas as pl
from jax.experimental.pallas import tpu as pltpu
from jax.experimental.pallas import tpu_sc as plsc

B = 16384
EMB = 32
V = 1000000
NTILES = 7813
NFULL = 7812
VPAD = NTILES * 128


def kernel(part_users, pos_items, neg_items, emb_users, emb_items):
    info = plsc.get_sparse_core_info()
    NC, NS = info.num_cores, info.num_subcores
    NW = NC * NS
    b_per_w = B // NW

    mesh = plsc.VectorSubcoreMesh(core_axis_name="c", subcore_axis_name="s")

    @functools.partial(
        pl.kernel,
        mesh=mesh,
        out_type=jax.ShapeDtypeStruct((NTILES * 32, 128), jnp.float32),
        compiler_params=pltpu.CompilerParams(
            use_tc_tiling_on_sc=True, needs_layout_passes=False
        ),
        scratch_types=[
            pltpu.VMEM((EMB, 512), jnp.float32),
            pltpu.VMEM((EMB, 512), jnp.float32),
            pltpu.VMEM((128, 128), jnp.float32),
            pltpu.VMEM((128, 128), jnp.float32),
            pltpu.VMEM((EMB, 64), jnp.float32),
            pltpu.SemaphoreType.DMA,
            pltpu.SemaphoreType.DMA,
            pltpu.SemaphoreType.DMA,
            pltpu.SemaphoreType.DMA,
        ],
    )
    def detile(tbl, out, ibuf0, ibuf1, obuf0, obuf1, tbuf,
               sem_i0, sem_i1, sem_o0, sem_o1):
        wid = lax.axis_index("s") * NC + lax.axis_index("c")
        iota16 = lax.iota(jnp.int32, 16)
        HI = [(32 * (16 * h + iota16)) // 128 for h in range(32)]
        LO = [lax.rem(32 * (16 * h + iota16), 128) for h in range(32)]
        ibufs, obufs = (ibuf0, ibuf1), (obuf0, obuf1)
        sems_i, sems_o = (sem_i0, sem_i1), (sem_o0, sem_o1)
        NSB = NFULL // 4
        nb = (NSB - wid + NW - 1) // NW

        def fetch(slot, i):
            sb = wid + NW * i
            off = pl.multiple_of(sb * 512, 128)
            pltpu.async_copy(
                tbl.at[:, pl.ds(off, 512)], ibufs[slot], sems_i[slot]
            )

        def rearrange(slot):
            src, dst = ibufs[slot], obufs[slot]

            def per_e(e, _):
                for h in range(32):
                    v = src[e, pl.ds(16 * h, 16)]
                    plsc.store_scatter(dst, [HI[h], LO[h] + e], v)
                return 0

            lax.fori_loop(0, EMB, per_e, 0)

        def drain_in(slot):
            pltpu.make_async_copy(
                tbl.at[:, pl.ds(0, 512)], ibufs[slot], sems_i[slot]
            ).wait()

        def drain_out(slot, i):
            sb = wid + NW * i
            pltpu.make_async_copy(
                obufs[slot], out.at[pl.ds(sb * 128, 128), :], sems_o[slot]
            ).wait()

        @pl.when(nb > 0)
        def _():
            fetch(0, 0)

        def body(i, _):
            slot = lax.rem(i, 2)
            for s in range(2):
                @pl.when(slot == s)
                def _():
                    @pl.when(i + 1 < nb)
                    def _():
                        fetch(1 - s, i + 1)
                    drain_in(s)
                    @pl.when(i >= 2)
                    def _():
                        drain_out(s, i - 2)
                    rearrange(s)
                    sb = wid + NW * i
                    pltpu.async_copy(
                        obufs[s], out.at[pl.ds(sb * 128, 128), :], sems_o[s]
                    )
            return 0

        lax.fori_loop(0, nb, body, 0)

        def tail_drain(i, _):
            for s in range(2):
                @pl.when((lax.rem(i, 2) == s) & (i >= 0))
                def _():
                    drain_out(s, i)
            return 0

        lax.fori_loop(lax.max(nb - 2, 0), nb, tail_drain, 0)

        @pl.when(wid == NW - 1)
        def _():
            pltpu.sync_copy(tbl.at[:, pl.ds(NFULL * 128, 64)], tbuf)
            for e in range(EMB):
                for h in range(4):
                    v = tbuf[e, pl.ds(16 * h, 16)]
                    plsc.store_scatter(obuf0, [HI[h], LO[h] + e], v)
            pltpu.sync_copy(obuf0.at[pl.ds(0, 16), :],
                            out.at[pl.ds(NFULL * 32, 16), :])

    row_t = jax.ShapeDtypeStruct((B, EMB), jnp.float32)

    @functools.partial(
        pl.kernel,
        mesh=mesh,
        out_type=[row_t, row_t, row_t],
        compiler_params=pltpu.CompilerParams(use_tc_tiling_on_sc=False),
        scratch_types=[
            pltpu.VMEM((b_per_w,), jnp.int32),
            pltpu.VMEM((b_per_w,), jnp.int32),
            pltpu.VMEM((b_per_w,), jnp.int32),
            pltpu.VMEM((b_per_w, EMB), jnp.float32),
            pltpu.VMEM((b_per_w, EMB), jnp.float32),
            pltpu.VMEM((b_per_w, EMB), jnp.float32),
            pltpu.SemaphoreType.DMA,
            pltpu.SemaphoreType.DMA,
            pltpu.SemaphoreType.DMA,
        ],
    )
    def gather3(pu_hbm, pi_hbm, ni_hbm, eu_hbm, ei_hbm,
                out_u, out_p, out_n,
                idx_u, idx_p, idx_n,
                rows_u, rows_p, rows_n,
                sem_u, sem_p, sem_n):
        wid = lax.axis_index("s") * NC + lax.axis_index("c")
        base = wid * b_per_w
        pltpu.sync_copy(pu_hbm.at[pl.ds(base, b_per_w)], idx_u)
        pltpu.sync_copy(pi_hbm.at[pl.ds(base, b_per_w)], idx_p)
        pltpu.sync_copy(ni_hbm.at[pl.ds(base, b_per_w)], idx_n)
        cu = pltpu.async_copy(eu_hbm.at[idx_u], rows_u, sem_u)
        cp = pltpu.async_copy(ei_hbm.at[idx_p], rows_p, sem_p)
        cn = pltpu.async_copy(ei_hbm.at[idx_n], rows_n, sem_n)
        cu.wait()
        pltpu.sync_copy(rows_u, out_u.at[pl.ds(base, b_per_w)])
        cp.wait()
        pltpu.sync_copy(rows_p, out_p.at[pl.ds(base, b_per_w)])
        cn.wait()
        pltpu.sync_copy(rows_n, out_n.at[pl.ds(base, b_per_w)])

    eu_lin = detile(emb_users.T).reshape(VPAD, EMB)
    ei_lin = detile(emb_items.T).reshape(VPAD, EMB)
    out = gather3(part_users, pos_items, neg_items, eu_lin, ei_lin)
    return tuple(out)


# speedup vs baseline: 1.4130x; 1.4130x over previous
import functools

import jax
import jax.numpy as jnp
from jax import lax
from jax.experimental import pallas as pl
from jax.experimental.pallas import tpu as pltpu
from jax.experimental.pallas import tpu_sc as plsc

B = 16384
EMB = 32


def kernel(part_users, pos_items, neg_items, emb_users, emb_items):
    info = plsc.get_sparse_core_info()
    NC, NS = info.num_cores, info.num_subcores
    NW = NC * NS
    b_per_w = B // NW

    mesh = plsc.VectorSubcoreMesh(core_axis_name="c", subcore_axis_name="s")
    row_t = jax.ShapeDtypeStruct((B, EMB), jnp.float32)

    @functools.partial(
        pl.kernel,
        mesh=mesh,
        out_type=[row_t, row_t, row_t],
        compiler_params=pltpu.CompilerParams(use_tc_tiling_on_sc=False),
        scratch_types=[
            pltpu.VMEM((b_per_w,), jnp.int32),
            pltpu.VMEM((b_per_w,), jnp.int32),
            pltpu.VMEM((b_per_w,), jnp.int32),
            pltpu.VMEM((b_per_w, EMB), jnp.float32),
            pltpu.VMEM((b_per_w, EMB), jnp.float32),
            pltpu.VMEM((b_per_w, EMB), jnp.float32),
            pltpu.SemaphoreType.DMA,
            pltpu.SemaphoreType.DMA,
            pltpu.SemaphoreType.DMA,
        ],
    )
    def gather3(pu_hbm, pi_hbm, ni_hbm, eu_hbm, ei_hbm,
                out_u, out_p, out_n,
                idx_u, idx_p, idx_n,
                rows_u, rows_p, rows_n,
                sem_u, sem_p, sem_n):
        wid = lax.axis_index("s") * NC + lax.axis_index("c")
        base = wid * b_per_w
        pltpu.sync_copy(pu_hbm.at[pl.ds(base, b_per_w)], idx_u)
        pltpu.sync_copy(pi_hbm.at[pl.ds(base, b_per_w)], idx_p)
        pltpu.sync_copy(ni_hbm.at[pl.ds(base, b_per_w)], idx_n)
        cu = pltpu.async_copy(eu_hbm.at[idx_u], rows_u, sem_u)
        cp = pltpu.async_copy(ei_hbm.at[idx_p], rows_p, sem_p)
        cn = pltpu.async_copy(ei_hbm.at[idx_n], rows_n, sem_n)
        cu.wait()
        pltpu.sync_copy(rows_u, out_u.at[pl.ds(base, b_per_w)])
        cp.wait()
        pltpu.sync_copy(rows_p, out_p.at[pl.ds(base, b_per_w)])
        cn.wait()
        pltpu.sync_copy(rows_n, out_n.at[pl.ds(base, b_per_w)])

    out = gather3(part_users, pos_items, neg_items, emb_users, emb_items)
    return tuple(out)
